# baseline (speedup 1.0000x reference)
import jax
import jax.numpy as jnp
from jax import lax
from jax.experimental import pallas as pl
from jax.experimental.pallas import tpu as pltpu
from jax.experimental.pallas import tpu_sc as plsc

VOCAB = 1000000
MAXLEN = 200
EMBED_DIM = 64
BATCH = 4096
SEQ = 200

NUM_WORKERS = 32
BSTRIPE = BATCH // NUM_WORKERS
NBUF = 4
OUTER = SEQ // NBUF


def _tec_body(xt_hbm, tok_hbm, post_hbm, out_hbm, idx_v, post_v, g_bufs,
              t_bufs, g_sems, o_sems):
    wid = lax.axis_index("s") * 2 + lax.axis_index("c")
    b0 = wid * BSTRIPE

    pltpu.sync_copy(xt_hbm.at[:, pl.ds(b0, BSTRIPE)], idx_v)
    pltpu.sync_copy(post_hbm, post_v)

    lanes = lax.iota(jnp.int32, 16)

    def gather_start(s, b):
        pltpu.make_async_copy(tok_hbm.at[idx_v.at[s]], g_bufs[b],
                              g_sems.at[b]).start()

    def gather_wait(s, b):
        pltpu.make_async_copy(tok_hbm.at[idx_v.at[s]], g_bufs[b],
                              g_sems.at[b]).wait()

    def out_copy(s, b):
        return pltpu.make_async_copy(t_bufs[b], out_hbm.at[s, :, wid],
                                     o_sems.at[b])

    for b in range(NBUF):
        gather_start(b, b)

    def outer_step(m, carry):
        for b in range(NBUF):
            s = m * NBUF + b
            gather_wait(s, b)
            g = g_bufs[b]
            t = t_bufs[b]
            s_splat = jnp.full((16,), s, jnp.int32)

            def per_d(d, _):
                dsp = jnp.full((16,), d, jnp.int32)
                pvec = plsc.load_gather(post_v, [dsp, s_splat])
                dt = d >> 3
                di = d & 7
                for c in range(BSTRIPE // 16):
                    rows = plsc.load_gather(g, [c * 16 + lanes, dsp])
                    t[dt, di, pl.ds(c * 16, 16)] = rows + pvec
                return _

            lax.fori_loop(0, EMBED_DIM, per_d, 0, unroll=4)

            gather_start(lax.rem(s + NBUF, SEQ), b)

            @pl.when(m > 0)
            def _():
                out_copy(s, b).wait()

            out_copy(s, b).start()
        return carry

    lax.fori_loop(0, OUTER, outer_step, 0)

    for b in range(NBUF):
        gather_wait(b, b)
        out_copy(SEQ - NBUF + b, b).wait()


@jax.jit
def _embed(xt, token_table, post):
    mesh = plsc.VectorSubcoreMesh(core_axis_name="c", subcore_axis_name="s")
    return pl.kernel(
        _tec_body,
        out_type=jax.ShapeDtypeStruct(
            (SEQ, EMBED_DIM // 8, NUM_WORKERS, 8, BSTRIPE), jnp.float32),
        mesh=mesh,
        scratch_types=[
            pltpu.VMEM((SEQ, BSTRIPE), jnp.int32),
            pltpu.VMEM((EMBED_DIM, SEQ), jnp.float32),
            [pltpu.VMEM((BSTRIPE, EMBED_DIM), jnp.float32)
             for _ in range(NBUF)],
            [pltpu.VMEM((EMBED_DIM // 8, 8, BSTRIPE), jnp.float32)
             for _ in range(NBUF)],
            pltpu.SemaphoreType.DMA((NBUF,)),
            pltpu.SemaphoreType.DMA((NBUF,)),
        ],
        compiler_params=pltpu.CompilerParams(use_tc_tiling_on_sc=False,
                                             needs_layout_passes=False),
    )(xt, token_table, post)


def kernel(x, token_table, pos_table):
    out5 = _embed(x.T, token_table, pos_table.T)
    return out5.transpose(2, 4, 0, 1, 3).reshape(BATCH, SEQ, EMBED_DIM)


# speedup vs baseline: 1.7520x; 1.7520x over previous
import jax
import jax.numpy as jnp
from jax import lax
from jax.experimental import pallas as pl
from jax.experimental.pallas import tpu as pltpu
from jax.experimental.pallas import tpu_sc as plsc

VOCAB = 1000000
MAXLEN = 200
EMBED_DIM = 64
BATCH = 4096
SEQ = 200

NUM_WORKERS = 32
BSTRIPE = BATCH // NUM_WORKERS
NBUF = 4
OUTER = SEQ // NBUF
TPITCH = BSTRIPE + 1


def _tec_body(xt_hbm, tok_hbm, pos_hbm, out_hbm, idx_v, pos_v, g_bufs,
              t_bufs, g_sems, o_sems):
    wid = lax.axis_index("s") * 2 + lax.axis_index("c")
    b0 = wid * BSTRIPE

    pltpu.sync_copy(xt_hbm.at[:, pl.ds(b0, BSTRIPE)], idx_v)
    pltpu.sync_copy(pos_hbm, pos_v)

    lanes = lax.iota(jnp.int32, 16)
    dt_c = [(jnp.int32(c * 16) + lanes) >> 3 for c in range(EMBED_DIM // 16)]
    di_c = [(jnp.int32(c * 16) + lanes) & 7 for c in range(EMBED_DIM // 16)]

    def gather_start(s, b):
        pltpu.make_async_copy(tok_hbm.at[idx_v.at[s]], g_bufs[b],
                              g_sems.at[b]).start()

    def gather_wait(s, b):
        pltpu.make_async_copy(tok_hbm.at[idx_v.at[s]], g_bufs[b],
                              g_sems.at[b]).wait()

    def out_copy(s, b):
        return pltpu.make_async_copy(
            t_bufs[b].at[:, :, pl.ds(0, BSTRIPE)], out_hbm.at[s, :, wid],
            o_sems.at[b])

    for b in range(NBUF):
        gather_start(b, b)

    def outer_step(m, carry):
        for b in range(NBUF):
            s = m * NBUF + b
            gather_wait(s, b)
            g = g_bufs[b]
            t = t_bufs[b]
            pv = [pos_v[s, pl.ds(c * 16, 16)] for c in range(EMBED_DIM // 16)]

            def per_row(r, _):
                rsp = jnp.full((16,), r, jnp.int32)
                for c in range(EMBED_DIM // 16):
                    vals = g[r, pl.ds(c * 16, 16)] + pv[c]
                    plsc.store_scatter(t, [dt_c[c], di_c[c], rsp], vals)
                return _

            lax.fori_loop(0, BSTRIPE, per_row, 0, unroll=8)

            gather_start(lax.rem(s + NBUF, SEQ), b)

            @pl.when(m > 0)
            def _():
                out_copy(s, b).wait()

            out_copy(s, b).start()
        return carry

    lax.fori_loop(0, OUTER, outer_step, 0)

    for b in range(NBUF):
        gather_wait(b, b)
        out_copy(SEQ - NBUF + b, b).wait()


@jax.jit
def _embed(xt, token_table, pos_table):
    mesh = plsc.VectorSubcoreMesh(core_axis_name="c", subcore_axis_name="s")
    return pl.kernel(
        _tec_body,
        out_type=jax.ShapeDtypeStruct(
            (SEQ, EMBED_DIM // 8, NUM_WORKERS, 8, BSTRIPE), jnp.float32),
        mesh=mesh,
        scratch_types=[
            pltpu.VMEM((SEQ, BSTRIPE), jnp.int32),
            pltpu.VMEM((MAXLEN, EMBED_DIM), jnp.float32),
            [pltpu.VMEM((BSTRIPE, EMBED_DIM), jnp.float32)
             for _ in range(NBUF)],
            [pltpu.VMEM((EMBED_DIM // 8, 8, TPITCH), jnp.float32)
             for _ in range(NBUF)],
            pltpu.SemaphoreType.DMA((NBUF,)),
            pltpu.SemaphoreType.DMA((NBUF,)),
        ],
        compiler_params=pltpu.CompilerParams(use_tc_tiling_on_sc=False,
                                             needs_layout_passes=False),
    )(xt, token_table, pos_table)


def kernel(x, token_table, pos_table):
    out5 = _embed(x.T, token_table, pos_table)
    return out5.transpose(2, 4, 0, 1, 3).reshape(BATCH, SEQ, EMBED_DIM)


# speedup vs baseline: 2.5326x; 1.4455x over previous
import jax
import jax.numpy as jnp
from jax import lax
from jax.experimental import pallas as pl
from jax.experimental.pallas import tpu as pltpu
from jax.experimental.pallas import tpu_sc as plsc

VOCAB = 1000000
MAXLEN = 200
EMBED_DIM = 64
BATCH = 4096
SEQ = 200

NUM_WORKERS = 32
BSTRIPE = BATCH // NUM_WORKERS
NBUF = 4
OUTER = SEQ // NBUF
TPITCH = BSTRIPE + 1


def _tec_body(xt_hbm, tok_hbm, pos_hbm, out_hbm, idx_v, pos_v, g_bufs,
              t_bufs, g_sems, o_sems):
    wid = lax.axis_index("s") * 2 + lax.axis_index("c")
    b0 = wid * BSTRIPE

    pltpu.sync_copy(xt_hbm.at[:, pl.ds(b0, BSTRIPE)], idx_v)
    pltpu.sync_copy(pos_hbm, pos_v)

    lanes = lax.iota(jnp.int32, 16)
    dt_c = [(jnp.int32(c * 16) + lanes) >> 3 for c in range(EMBED_DIM // 16)]
    di_c = [(jnp.int32(c * 16) + lanes) & 7 for c in range(EMBED_DIM // 16)]

    def gather_start(s, b):
        pltpu.make_async_copy(tok_hbm.at[idx_v.at[s]], g_bufs[b],
                              g_sems.at[b]).start()

    def gather_wait(s, b):
        pltpu.make_async_copy(tok_hbm.at[idx_v.at[s]], g_bufs[b],
                              g_sems.at[b]).wait()

    def out_copy(s, b):
        return pltpu.make_async_copy(
            t_bufs[b].at[:, :, pl.ds(0, BSTRIPE)], out_hbm.at[s, :, wid],
            o_sems.at[b])

    for b in range(NBUF):
        gather_start(b, b)

    def outer_step(m, carry):
        for b in range(NBUF):
            s = m * NBUF + b
            gather_wait(s, b)
            g = g_bufs[b]
            t = t_bufs[b]
            pv = [pos_v[s, pl.ds(c * 16, 16)] for c in range(EMBED_DIM // 16)]

            @plsc.parallel_loop(0, BSTRIPE, unroll=8)
            def per_row(r):
                rsp = jnp.full((16,), r, jnp.int32)
                for c in range(EMBED_DIM // 16):
                    vals = g[r, pl.ds(c * 16, 16)] + pv[c]
                    plsc.store_scatter(t, [dt_c[c], di_c[c], rsp], vals)

            gather_start(lax.rem(s + NBUF, SEQ), b)

            @pl.when(m > 0)
            def _():
                out_copy(s, b).wait()

            out_copy(s, b).start()
        return carry

    lax.fori_loop(0, OUTER, outer_step, 0)

    for b in range(NBUF):
        gather_wait(b, b)
        out_copy(SEQ - NBUF + b, b).wait()


@jax.jit
def _embed(xt, token_table, pos_table):
    mesh = plsc.VectorSubcoreMesh(core_axis_name="c", subcore_axis_name="s")
    return pl.kernel(
        _tec_body,
        out_type=jax.ShapeDtypeStruct(
            (SEQ, EMBED_DIM // 8, NUM_WORKERS, 8, BSTRIPE), jnp.float32),
        mesh=mesh,
        scratch_types=[
            pltpu.VMEM((SEQ, BSTRIPE), jnp.int32),
            pltpu.VMEM((MAXLEN, EMBED_DIM), jnp.float32),
            [pltpu.VMEM((BSTRIPE, EMBED_DIM), jnp.float32)
             for _ in range(NBUF)],
            [pltpu.VMEM((EMBED_DIM // 8, 8, TPITCH), jnp.float32)
             for _ in range(NBUF)],
            pltpu.SemaphoreType.DMA((NBUF,)),
            pltpu.SemaphoreType.DMA((NBUF,)),
        ],
        compiler_params=pltpu.CompilerParams(use_tc_tiling_on_sc=False,
                                             needs_layout_passes=False),
    )(xt, token_table, pos_table)


def kernel(x, token_table, pos_table):
    out5 = _embed(x.T, token_table, pos_table)
    return out5.transpose(2, 4, 0, 1, 3).reshape(BATCH, SEQ, EMBED_DIM)


# speedup vs baseline: 3.7881x; 1.4957x over previous
import jax
import jax.numpy as jnp
from jax import lax
from jax.experimental import pallas as pl
from jax.experimental import layout as jex_layout
from jax.experimental.pallas import tpu as pltpu
from jax.experimental.pallas import tpu_sc as plsc

VOCAB = 1000000
MAXLEN = 200
EMBED_DIM = 64
BATCH = 4096
SEQ = 200

NUM_WORKERS = 32
BSTRIPE = BATCH // NUM_WORKERS
NBUF = 4
OUTER = SEQ // NBUF
TPITCH = BSTRIPE + 1


def _tec_body(xt_hbm, tok_hbm, pos_hbm, out_hbm, idx_v, pos_v, g_bufs,
              t_bufs, g_sems, o_sems):
    wid = lax.axis_index("s") * 2 + lax.axis_index("c")
    b0 = wid * BSTRIPE

    pltpu.sync_copy(xt_hbm.at[:, pl.ds(b0, BSTRIPE)], idx_v)
    pltpu.sync_copy(pos_hbm, pos_v)

    lanes = lax.iota(jnp.int32, 16)
    dt_c = [(jnp.int32(c * 16) + lanes) >> 3 for c in range(EMBED_DIM // 16)]
    di_c = [(jnp.int32(c * 16) + lanes) & 7 for c in range(EMBED_DIM // 16)]

    def gather_start(s, b):
        pltpu.make_async_copy(tok_hbm.at[idx_v.at[s]], g_bufs[b],
                              g_sems.at[b]).start()

    def gather_wait(s, b):
        pltpu.make_async_copy(tok_hbm.at[idx_v.at[s]], g_bufs[b],
                              g_sems.at[b]).wait()

    def out_copy(s, b):
        return pltpu.make_async_copy(
            t_bufs[b].at[:, :, pl.ds(0, BSTRIPE)], out_hbm.at[s, :, wid],
            o_sems.at[b])

    for b in range(NBUF):
        gather_start(b, b)

    def outer_step(m, carry):
        for b in range(NBUF):
            s = m * NBUF + b
            gather_wait(s, b)
            g = g_bufs[b]
            t = t_bufs[b]
            pv = [pos_v[s, pl.ds(c * 16, 16)] for c in range(EMBED_DIM // 16)]

            @plsc.parallel_loop(0, BSTRIPE, unroll=8)
            def per_row(r):
                rsp = jnp.full((16,), r, jnp.int32)
                for c in range(EMBED_DIM // 16):
                    vals = g[r, pl.ds(c * 16, 16)] + pv[c]
                    plsc.store_scatter(t, [dt_c[c], di_c[c], rsp], vals)

            gather_start(lax.rem(s + NBUF, SEQ), b)

            @pl.when(m > 0)
            def _():
                out_copy(s, b).wait()

            out_copy(s, b).start()
        return carry

    lax.fori_loop(0, OUTER, outer_step, 0)

    for b in range(NBUF):
        gather_wait(b, b)
        out_copy(SEQ - NBUF + b, b).wait()


@jax.jit
def _embed(xt, token_table, pos_table):
    mesh = plsc.VectorSubcoreMesh(core_axis_name="c", subcore_axis_name="s")
    return pl.kernel(
        _tec_body,
        out_type=jax.ShapeDtypeStruct(
            (SEQ, EMBED_DIM // 8, NUM_WORKERS, 8, BSTRIPE), jnp.float32),
        mesh=mesh,
        scratch_types=[
            pltpu.VMEM((SEQ, BSTRIPE), jnp.int32),
            pltpu.VMEM((MAXLEN, EMBED_DIM), jnp.float32),
            [pltpu.VMEM((BSTRIPE, EMBED_DIM), jnp.float32)
             for _ in range(NBUF)],
            [pltpu.VMEM((EMBED_DIM // 8, 8, TPITCH), jnp.float32)
             for _ in range(NBUF)],
            pltpu.SemaphoreType.DMA((NBUF,)),
            pltpu.SemaphoreType.DMA((NBUF,)),
        ],
        compiler_params=pltpu.CompilerParams(use_tc_tiling_on_sc=False,
                                             needs_layout_passes=False),
    )(xt, token_table, pos_table)


def kernel(x, token_table, pos_table):
    token_table = jex_layout.with_layout_constraint(
        token_table, jex_layout.Layout(major_to_minor=(0, 1), tiling=((8,),)))
    out5 = _embed(x.T, token_table, pos_table)
    return out5.transpose(2, 4, 0, 1, 3).reshape(BATCH, SEQ, EMBED_DIM)


# speedup vs baseline: 3.8017x; 1.0036x over previous
import jax
import jax.numpy as jnp
from jax import lax
from jax.experimental import pallas as pl
from jax.experimental import layout as jex_layout
from jax.experimental.pallas import tpu as pltpu
from jax.experimental.pallas import tpu_sc as plsc

VOCAB = 1000000
MAXLEN = 200
EMBED_DIM = 64
BATCH = 4096
SEQ = 200

NUM_WORKERS = 32
BSTRIPE = BATCH // NUM_WORKERS
NBUF = 4
OUTER = SEQ // NBUF
TPITCH = BSTRIPE + 1


def _tec_body(xt_hbm, tok_hbm, pos_hbm, out_hbm, idx_v, pos_v, g_bufs,
              t_bufs, g_sems, o_sems):
    wid = lax.axis_index("s") * 2 + lax.axis_index("c")
    b0 = wid * BSTRIPE

    pltpu.sync_copy(xt_hbm.at[:, pl.ds(b0, BSTRIPE)], idx_v)
    pltpu.sync_copy(pos_hbm, pos_v)

    @plsc.parallel_loop(0, SEQ, unroll=4)
    def _dbl(s):
        for c in range(BSTRIPE // 16):
            sl = pl.ds(c * 16, 16)
            idx_v[s, sl] = idx_v[s, sl] + idx_v[s, sl]

    lanes = lax.iota(jnp.int32, 16)
    dt_c = [(jnp.int32(c * 16) + lanes) >> 3 for c in range(EMBED_DIM // 16)]
    di_c = [(jnp.int32(c * 16) + lanes) & 7 for c in range(EMBED_DIM // 16)]

    def gather_start(s, b):
        pltpu.make_async_copy(tok_hbm.at[idx_v.at[s]], g_bufs[b],
                              g_sems.at[b]).start()

    def gather_wait(s, b):
        pltpu.make_async_copy(tok_hbm.at[idx_v.at[s]], g_bufs[b],
                              g_sems.at[b]).wait()

    def out_copy(s, b):
        return pltpu.make_async_copy(
            t_bufs[b].at[:, :, pl.ds(0, BSTRIPE)], out_hbm.at[s, :, wid],
            o_sems.at[b])

    for b in range(NBUF):
        gather_start(b, b)

    def outer_step(m, carry):
        for b in range(NBUF):
            s = m * NBUF + b
            gather_wait(s, b)
            g = g_bufs[b]
            t = t_bufs[b]
            pv = [pos_v[s, pl.ds(c * 16, 16)] for c in range(EMBED_DIM // 16)]

            @plsc.parallel_loop(0, BSTRIPE, unroll=8)
            def per_row(r):
                rsp = jnp.full((16,), r, jnp.int32)
                for c in range(EMBED_DIM // 16):
                    vals = g[r, pl.ds(c * 16, 16)] + pv[c]
                    plsc.store_scatter(t, [dt_c[c], di_c[c], rsp], vals)

            gather_start(lax.rem(s + NBUF, SEQ), b)

            @pl.when(m > 0)
            def _():
                out_copy(s, b).wait()

            out_copy(s, b).start()
        return carry

    lax.fori_loop(0, OUTER, outer_step, 0)

    for b in range(NBUF):
        gather_wait(b, b)
        out_copy(SEQ - NBUF + b, b).wait()


@jax.jit
def _embed(xt, token_table, pos_table):
    mesh = plsc.VectorSubcoreMesh(core_axis_name="c", subcore_axis_name="s")
    return pl.kernel(
        _tec_body,
        out_type=jax.ShapeDtypeStruct(
            (SEQ, EMBED_DIM // 8, NUM_WORKERS, 8, BSTRIPE), jnp.float32),
        mesh=mesh,
        scratch_types=[
            pltpu.VMEM((SEQ, BSTRIPE), jnp.int32),
            pltpu.VMEM((MAXLEN, EMBED_DIM), jnp.float32),
            [pltpu.VMEM((BSTRIPE, EMBED_DIM), jnp.float32)
             for _ in range(NBUF)],
            [pltpu.VMEM((EMBED_DIM // 8, 8, TPITCH), jnp.float32)
             for _ in range(NBUF)],
            pltpu.SemaphoreType.DMA((NBUF,)),
            pltpu.SemaphoreType.DMA((NBUF,)),
        ],
        compiler_params=pltpu.CompilerParams(use_tc_tiling_on_sc=False,
                                             needs_layout_passes=False),
    )(xt, token_table, pos_table)


def kernel(x, token_table, pos_table):
    token_table = jex_layout.with_layout_constraint(
        token_table,
        jex_layout.Layout(major_to_minor=(0, 1), tiling=((8, 128),)))
    out5 = _embed(x.T, token_table, pos_table)
    return out5.transpose(2, 4, 0, 1, 3).reshape(BATCH, SEQ, EMBED_DIM)
